# Initial kernel scaffold; baseline (speedup 1.0000x reference)
import jax
import jax.numpy as jnp
from jax.experimental import pallas as pl


def kernel(x, weight, weight_rc):
    raise NotImplementedError("write your pallas kernel here")


# speedup vs baseline: 30.7963x; 30.7963x over previous
import functools

import jax
import jax.numpy as jnp
from jax import lax
from jax.experimental import pallas as pl
from jax.experimental.pallas import tpu as pltpu
from jax.experimental.pallas import tpu_sc as plsc

B = 4096
L = 200
NUM_EMB = 4
G = 8
GV = G * L // 16
NW = 32
ROWS_PER_W = B // NW
GROUPS_PER_W = ROWS_PER_W // G


def _sc_embed(x_flat, wcols):
    mesh = plsc.VectorSubcoreMesh(core_axis_name="c", subcore_axis_name="s")

    @functools.partial(
        pl.kernel,
        mesh=mesh,
        out_type=jax.ShapeDtypeStruct((2 * B * NUM_EMB * L,), jnp.float32),
        scratch_types=[
            pltpu.VMEM((G * L,), jnp.int32),
            pltpu.VMEM((G * L,), jnp.int32),
            pltpu.VMEM((G * NUM_EMB * L,), jnp.float32),
            pltpu.VMEM((G * NUM_EMB * L,), jnp.float32),
            pltpu.VMEM((2 * (NUM_EMB + 1) * NUM_EMB,), jnp.float32),
            pltpu.SemaphoreType.DMA,
            pltpu.SemaphoreType.DMA,
        ],
        compiler_params=pltpu.CompilerParams(needs_layout_passes=False),
    )
    def k(x_hbm, wcols_hbm, out_hbm, xv0, xv1, slab_f, slab_r, luts, sem0,
          sem1):
        wid = lax.axis_index("s") * 2 + lax.axis_index("c")
        row0 = wid * ROWS_PER_W
        pltpu.sync_copy(wcols_hbm, luts)
        iota = lax.iota(jnp.int32, 16)
        bufs = (xv0, xv1)
        sems = (sem0, sem1)

        pltpu.async_copy(x_hbm.at[pl.ds(row0 * L, G * L)], xv0, sem0)

        def pair_body(s, _):
            for par in range(2):
                g = 2 * s + par

                @pl.when(g + 1 < GROUPS_PER_W)
                def _(par=par, g=g):
                    nxt = (row0 + (g + 1) * G) * L
                    pltpu.async_copy(
                        x_hbm.at[pl.ds(nxt, G * L)], bufs[1 - par],
                        sems[1 - par])

                pltpu.make_async_copy(
                    x_hbm.at[pl.ds(row0 * L, G * L)], bufs[par],
                    sems[par]).wait()

                def vec_body(j, _, par=par):
                    base = j * 16
                    v = bufs[par][pl.ds(base, 16)]
                    p = iota + base
                    rloc = lax.shift_right_logical(p * 5243, 20)
                    dst_f = p + 600 * rloc
                    dst_r = 1000 * rloc + (199 - p)
                    for e in range(NUM_EMB):
                        val_f = plsc.load_gather(luts, [v + (5 * e)])
                        plsc.store_scatter(slab_f, [dst_f + (200 * e)], val_f)
                        val_r = plsc.load_gather(luts, [v + (20 + 5 * e)])
                        plsc.store_scatter(slab_r, [dst_r + (200 * e)], val_r)
                    return 0

                lax.fori_loop(0, GV, vec_body, 0)
                out_f = (row0 + g * G) * (NUM_EMB * L)
                out_r = (B + row0 + g * G) * (NUM_EMB * L)
                pltpu.sync_copy(
                    slab_f, out_hbm.at[pl.ds(out_f, G * NUM_EMB * L)])
                pltpu.sync_copy(
                    slab_r, out_hbm.at[pl.ds(out_r, G * NUM_EMB * L)])
            return 0

        lax.fori_loop(0, GROUPS_PER_W // 2, pair_body, 0)

    return k(x_flat, wcols)


def kernel(x, weight, weight_rc):
    x_flat = x.astype(jnp.int32).reshape(-1)
    wcols = jnp.concatenate(
        [weight.T.reshape(-1), weight_rc.T.reshape(-1)]).astype(jnp.float32)
    out = _sc_embed(x_flat, wcols)
    return out.reshape(2 * B, NUM_EMB, L)


# speedup vs baseline: 33.4821x; 1.0872x over previous
import functools

import jax
import jax.numpy as jnp
from jax import lax
from jax.experimental import pallas as pl
from jax.experimental.pallas import tpu as pltpu
from jax.experimental.pallas import tpu_sc as plsc

B = 4096
L = 200
NUM_EMB = 4
G = 16
GV = G * L // 16
SLAB = G * NUM_EMB * L
NW = 32
ROWS_PER_W = B // NW
GROUPS_PER_W = ROWS_PER_W // G


def _sc_embed(x_flat, wcols):
    mesh = plsc.VectorSubcoreMesh(core_axis_name="c", subcore_axis_name="s")

    @functools.partial(
        pl.kernel,
        mesh=mesh,
        out_type=jax.ShapeDtypeStruct((2 * B * NUM_EMB * L,), jnp.float32),
        scratch_types=[
            pltpu.VMEM((G * L,), jnp.int32),
            pltpu.VMEM((G * L,), jnp.int32),
            pltpu.VMEM((SLAB,), jnp.float32),
            pltpu.VMEM((SLAB,), jnp.float32),
            pltpu.VMEM((SLAB,), jnp.float32),
            pltpu.VMEM((SLAB,), jnp.float32),
            pltpu.VMEM((2 * (NUM_EMB + 1) * NUM_EMB,), jnp.float32),
            pltpu.SemaphoreType.DMA,
            pltpu.SemaphoreType.DMA,
            pltpu.SemaphoreType.DMA,
            pltpu.SemaphoreType.DMA,
            pltpu.SemaphoreType.DMA,
            pltpu.SemaphoreType.DMA,
        ],
        compiler_params=pltpu.CompilerParams(needs_layout_passes=False),
    )
    def k(x_hbm, wcols_hbm, out_hbm, xv0, xv1, sf0, sf1, sr0, sr1, luts,
          semx0, semx1, semf0, semf1, semr0, semr1):
        wid = lax.axis_index("s") * 2 + lax.axis_index("c")
        row0 = wid * ROWS_PER_W
        pltpu.sync_copy(wcols_hbm, luts)
        iota = lax.iota(jnp.int32, 16)
        xbufs = (xv0, xv1)
        fslabs = (sf0, sf1)
        rslabs = (sr0, sr1)
        semx = (semx0, semx1)
        semf = (semf0, semf1)
        semr = (semr0, semr1)

        pltpu.async_copy(x_hbm.at[pl.ds(row0 * L, G * L)], xv0, semx0)

        def pair_body(s, _):
            for par in range(2):
                g = 2 * s + par
                slab_f = fslabs[par]
                slab_r = rslabs[par]

                @pl.when(g + 1 < GROUPS_PER_W)
                def _(par=par, g=g):
                    nxt = (row0 + (g + 1) * G) * L
                    pltpu.async_copy(
                        x_hbm.at[pl.ds(nxt, G * L)], xbufs[1 - par],
                        semx[1 - par])

                pltpu.make_async_copy(
                    x_hbm.at[pl.ds(row0 * L, G * L)], xbufs[par],
                    semx[par]).wait()

                @pl.when(g >= 2)
                def _(par=par):
                    pltpu.make_async_copy(
                        fslabs[par], out_hbm.at[pl.ds(0, SLAB)],
                        semf[par]).wait()
                    pltpu.make_async_copy(
                        rslabs[par], out_hbm.at[pl.ds(0, SLAB)],
                        semr[par]).wait()

                def vec_body(j, _, par=par, slab_f=slab_f, slab_r=slab_r):
                    for u in range(2):
                        base = j * 32 + u * 16
                        v = xbufs[par][pl.ds(base, 16)]
                        p = iota + base
                        rloc = lax.shift_right_logical(p * 5243, 20)
                        dst_f = p + 600 * rloc
                        dst_r = 1000 * rloc + (199 - p)
                        for e in range(NUM_EMB):
                            val_f = plsc.load_gather(luts, [v + (5 * e)])
                            plsc.store_scatter(
                                slab_f, [dst_f + (200 * e)], val_f)
                            val_r = plsc.load_gather(luts, [v + (20 + 5 * e)])
                            plsc.store_scatter(
                                slab_r, [dst_r + (200 * e)], val_r)
                    return 0

                lax.fori_loop(0, GV // 2, vec_body, 0)
                out_f = (row0 + g * G) * (NUM_EMB * L)
                out_r = (B + row0 + g * G) * (NUM_EMB * L)
                pltpu.async_copy(
                    slab_f, out_hbm.at[pl.ds(out_f, SLAB)], semf[par])
                pltpu.async_copy(
                    slab_r, out_hbm.at[pl.ds(out_r, SLAB)], semr[par])
            return 0

        lax.fori_loop(0, GROUPS_PER_W // 2, pair_body, 0)

        for par in range(2):
            pltpu.make_async_copy(
                fslabs[par], out_hbm.at[pl.ds(0, SLAB)], semf[par]).wait()
            pltpu.make_async_copy(
                rslabs[par], out_hbm.at[pl.ds(0, SLAB)], semr[par]).wait()

    return k(x_flat, wcols)


def kernel(x, weight, weight_rc):
    x_flat = x.astype(jnp.int32).reshape(-1)
    wcols = jnp.concatenate(
        [weight.T.reshape(-1), weight_rc.T.reshape(-1)]).astype(jnp.float32)
    out = _sc_embed(x_flat, wcols)
    return out.reshape(2 * B, NUM_EMB, L)
